# Initial kernel scaffold; baseline (speedup 1.0000x reference)
import jax
import jax.numpy as jnp
from jax.experimental import pallas as pl


def kernel(x, edge_index, W1, b1, W2, b2):
    raise NotImplementedError("write your pallas kernel here")


# speedup vs baseline: 5.1864x; 5.1864x over previous
import functools

import jax
import jax.numpy as jnp
from jax import lax
from jax.experimental import pallas as pl
from jax.experimental.pallas import tpu as pltpu
from jax.experimental.pallas import tpu_sc as plsc

N = 10000
E = 320000
D = 128
DH = D // 2

NC = 2
NS = 16
C = 128
CHUNKS = 160
EPT = C * CHUNKS
E_PAD = NS * EPT
N_ACC = 10112
RPT = N_ACC // NS
DEGW = 8


def _sc_aggregate(with_deg):
    mesh = plsc.VectorSubcoreMesh(core_axis_name="c", subcore_axis_name="s")

    out_type = [
        jax.ShapeDtypeStruct((NC, N_ACC, DH), jnp.float32),
    ]
    if with_deg:
        out_type.append(jax.ShapeDtypeStruct((NC, N_ACC, DEGW), jnp.float32))

    scratch = [
        pltpu.VMEM((CHUNKS, C), jnp.int32),
        pltpu.VMEM((CHUNKS, C), jnp.int32),
        pltpu.VMEM((C, DH), jnp.float32),
        pltpu.VMEM((C, DH), jnp.float32),
        pltpu.VMEM_SHARED((N_ACC, DH), jnp.float32),
        pltpu.SemaphoreType.DMA,
        pltpu.SemaphoreType.DMA,
    ]
    if with_deg:
        scratch.append(pltpu.VMEM((C, DEGW), jnp.float32))
        scratch.append(pltpu.VMEM_SHARED((N_ACC, DEGW), jnp.float32))

    def body(h2_hbm, src_hbm, dst_hbm, z1_hbm, z2_hbm, ones_hbm,
             p_out, *rest):
        if with_deg:
            (deg_out, src_v, dst_v, rows0, rows1, accum, sem0, sem1,
             ones_v, dega) = rest
        else:
            (src_v, dst_v, rows0, rows1, accum, sem0, sem1) = rest
        core = lax.axis_index("c")
        sub = lax.axis_index("s")
        rows = (rows0, rows1)
        sems = (sem0, sem1)
        table = h2_hbm.at[core]

        r0 = pl.multiple_of(sub * RPT, 8)
        pltpu.sync_copy(z1_hbm, accum.at[pl.ds(r0, RPT)])
        if with_deg:
            pltpu.sync_copy(z2_hbm, dega.at[pl.ds(r0, RPT)])
            pltpu.sync_copy(ones_hbm, ones_v)
        pltpu.sync_copy(src_hbm.at[sub], src_v)
        pltpu.sync_copy(dst_hbm.at[sub], dst_v)
        plsc.subcore_barrier()

        def issue(c_idx, b):
            pltpu.async_copy(table.at[src_v.at[c_idx]], rows[b], sems[b])

        def wait(c_idx, b):
            pltpu.make_async_copy(table.at[src_v.at[c_idx]], rows[b],
                                  sems[b]).wait()

        def scatter(c_idx, b):
            pltpu.sync_copy(rows[b], accum.at[dst_v.at[c_idx]], add=True)
            if with_deg:
                pltpu.sync_copy(ones_v, dega.at[dst_v.at[c_idx]], add=True)

        issue(0, 0)
        issue(1, 1)

        def loop_body(g, carry):
            for b in range(2):
                c_idx = g * 2 + b
                wait(c_idx, b)
                issue(c_idx + 2, b)
                scatter(c_idx, b)
            return carry

        lax.fori_loop(0, (CHUNKS - 2) // 2, loop_body, 0, unroll=2)
        for b in range(2):
            c_idx = CHUNKS - 2 + b
            wait(c_idx, b)
            scatter(c_idx, b)

        plsc.subcore_barrier()
        pltpu.sync_copy(accum.at[pl.ds(r0, RPT)],
                        p_out.at[core, pl.ds(r0, RPT)])
        if with_deg:
            pltpu.sync_copy(dega.at[pl.ds(r0, RPT)],
                            deg_out.at[core, pl.ds(r0, RPT)])

    return pl.kernel(body, out_type=out_type, mesh=mesh,
                     scratch_types=scratch,
                     compiler_params=pltpu.CompilerParams(
                         use_tc_tiling_on_sc=False))


_sc_layer1 = _sc_aggregate(with_deg=True)
_sc_layer2 = _sc_aggregate(with_deg=False)


def _tc_combine_body(relu, split_out, p_ref, deg_ref, w_ref, b_ref, o_ref):
    deg = deg_ref[0, :N, 0:1]
    deg = jnp.maximum(deg, 1.0)
    acc = (jnp.dot(p_ref[0, :N], w_ref[:DH],
                   preferred_element_type=jnp.float32)
           + jnp.dot(p_ref[1, :N], w_ref[DH:],
                     preferred_element_type=jnp.float32))
    res = acc / deg + b_ref[...]
    if relu:
        res = jnp.maximum(res, 0.0)
    if split_out:
        o_ref[0] = res[:, :DH]
        o_ref[1] = res[:, DH:]
    else:
        o_ref[...] = res


def _tc_combine(p, deg, w, b, relu, split_out):
    if split_out:
        out_shape = jax.ShapeDtypeStruct((NC, N, DH), jnp.float32)
    else:
        out_shape = jax.ShapeDtypeStruct((N, D), jnp.float32)
    return pl.pallas_call(
        functools.partial(_tc_combine_body, relu, split_out),
        out_shape=out_shape,
    )(p, deg, w, b.reshape(1, D))


def kernel(x, edge_index, W1, b1, W2, b2):
    src = edge_index[0]
    dst = edge_index[1]
    npad = E_PAD - E
    src_p = jnp.concatenate([src, jnp.zeros((npad,), jnp.int32)])
    dst_p = jnp.concatenate(
        [dst, N + (jnp.arange(npad, dtype=jnp.int32) % (N_ACC - N))])
    src3 = src_p.reshape(NS, CHUNKS, C)
    dst3 = dst_p.reshape(NS, CHUNKS, C)

    x2 = jnp.stack([x[:, :DH], x[:, DH:]])
    z1 = jnp.zeros((RPT, DH), jnp.float32)
    z2 = jnp.zeros((RPT, DEGW), jnp.float32)
    ones = jnp.ones((C, DEGW), jnp.float32)

    p1, deg = _sc_layer1(x2, src3, dst3, z1, z2, ones)
    h2 = _tc_combine(p1, deg, W1, b1, relu=True, split_out=True)
    (p2,) = _sc_layer2(h2, src3, dst3, z1, z2, ones)
    out = _tc_combine(p2, deg, W2, b2, relu=False, split_out=False)
    return out
